# baseline (speedup 1.0000x reference)
import functools

import jax
import jax.numpy as jnp
from jax.experimental import pallas as pl
from jax.experimental.pallas import tpu as pltpu

_ZDIM = 128
_BLOCKS = [((32, 2), (1024, 0.1, 32, (32, 32))),
           ((32, 1), (256, 0.2, 32, (32, 64))),
           ((32, 1), (128, 0.4, 32, (64, _ZDIM)))]

_IMAX = 2147483647
_QBITS = 19


def _sa_body(nconv, K, r2, N, Mt, C1, C2, *refs):
    nw = 2 * nconv + 5
    (xyz_nc_ref, xyz_cn_ref, feats_ref, ctr_ref) = refs[:4]
    wrefs = refs[4:4 + nw]
    out_ref = refs[4 + nw]
    scores_ref = refs[5 + nw]
    a_ref = refs[6 + nw]
    acc_ref = refs[7 + nw]

    W1xT = wrefs[2 * nconv][...]
    b1 = wrefs[2 * nconv + 2][...]
    W2T = wrefs[2 * nconv + 3][...]
    b2 = wrefs[2 * nconv + 4][...]

    Nc = N // 128

    @pl.when(pl.program_id(1) == 0)
    def _compute_a():
        x_nc = xyz_nc_ref[0]
        f = feats_ref[0]
        for i in range(nconv):
            WcT = wrefs[2 * i][...]
            bc = wrefs[2 * i + 1][...]
            f = jnp.maximum(
                jnp.dot(f, WcT, preferred_element_type=jnp.float32) + bc, 0.0)
        W1fT = wrefs[2 * nconv + 1][...]
        a = (jnp.dot(x_nc, W1xT, preferred_element_type=jnp.float32)
             + jnp.dot(f, W1fT, preferred_element_type=jnp.float32)
             + b1)
        a_ref[...] = jnp.concatenate(
            [a[c * 128:(c + 1) * 128, :] for c in range(Nc)], axis=1)
    x_cn = xyz_cn_ref[0]
    ctr = ctr_ref[0]
    U = -jnp.dot(ctr, W1xT, preferred_element_type=jnp.float32)

    x2 = jnp.sum(x_cn * x_cn, axis=0, keepdims=True)
    c2 = jnp.sum(ctr * ctr, axis=1, keepdims=True)
    d2 = (c2 + x2
          - 2.0 * jnp.dot(ctr, x_cn, preferred_element_type=jnp.float32))

    iota = jax.lax.broadcasted_iota(jnp.int32, (Mt, N), 1)
    fbmin = jnp.min(d2, axis=1, keepdims=True)
    fb_amin = jnp.min(jnp.where(d2 == fbmin, iota, N), axis=1, keepdims=True)

    q = (d2 * (float(2 ** _QBITS) / r2)).astype(jnp.int32)
    q = jnp.minimum(jnp.maximum(q, 0), 2 ** _QBITS - 2)
    packed = q * 4096 + iota
    scores_ref[...] = jnp.where(d2 <= r2, packed, _IMAX)

    A2 = a_ref[...].astype(jnp.bfloat16)
    iota128 = jax.lax.broadcasted_iota(jnp.int32, (Mt, 128), 1)
    iotab = jax.lax.broadcasted_iota(jnp.int32, (Mt, Nc * C1), 1)
    l2c1 = C1.bit_length() - 1
    Rm = jnp.where(
        jnp.bitwise_and(
            jax.lax.broadcasted_iota(jnp.int32, (Nc * C1, C1), 0), C1 - 1)
        == jax.lax.broadcasted_iota(jnp.int32, (Nc * C1, C1), 1),
        1.0, 0.0).astype(jnp.bfloat16)
    acc_ref[...] = jnp.zeros((Mt, C2), jnp.float32)

    def cond(carry):
        j, _, alive = carry
        return jnp.logical_and(j < K, alive)

    def body(carry):
        j, prev, _ = carry
        s = scores_ref[...]
        vmin = jnp.min(jnp.where(s > prev, s, _IMAX), axis=1, keepdims=True)
        valid = vmin < _IMAX
        idx_sel = jnp.where(valid, jnp.bitwise_and(vmin, 4095), fb_amin)
        lane = jnp.bitwise_and(idx_sel, 127)
        chunk = jnp.right_shift(idx_sel, 7)
        onehotL = jnp.where(iota128 == lane, 1.0, 0.0).astype(jnp.bfloat16)
        B2 = jnp.dot(onehotL, A2, preferred_element_type=jnp.float32)
        mask2 = jnp.right_shift(iotab, l2c1) == chunk
        mB = jnp.where(mask2, B2, 0.0).astype(jnp.bfloat16)
        g = jnp.dot(mB, Rm, preferred_element_type=jnp.float32)
        h = jnp.maximum(g + U, 0.0)
        h2 = jnp.maximum(
            jnp.dot(h, W2T, preferred_element_type=jnp.float32) + b2, 0.0)
        acc_ref[...] = jnp.maximum(acc_ref[...], h2)
        return j + 1, vmin, jnp.min(vmin[:, 0]) < _IMAX

    jax.lax.while_loop(
        cond, body, (0, jnp.full((Mt, 1), -1, jnp.int32), True))
    out_ref[0] = acc_ref[...]


def _sa_block(xyz_nc, xyz_cn, feats, ctr, convs, W1, b1, W2, b2, M, K, r2):
    B, N, _ = xyz_nc.shape
    C = feats.shape[-1]
    C1 = W1.shape[0]
    C2 = W2.shape[0]
    Mt = min(M, 128)

    ins = [xyz_nc, xyz_cn, feats, ctr]
    in_specs = [
        pl.BlockSpec((1, N, 3), lambda b, t: (b, 0, 0)),
        pl.BlockSpec((1, 3, N), lambda b, t: (b, 0, 0)),
        pl.BlockSpec((1, N, C), lambda b, t: (b, 0, 0)),
        pl.BlockSpec((1, Mt, 3), lambda b, t: (b, t, 0)),
    ]
    weights = []
    for (Wc, bc) in convs:
        weights += [Wc.T, bc.reshape(1, -1)]
    weights += [W1[:, :3].T, W1[:, 3:].T, b1.reshape(1, -1),
                W2.T, b2.reshape(1, -1)]
    for w in weights:
        ins.append(w)
        in_specs.append(pl.BlockSpec(w.shape, lambda b, t: (0, 0)))

    body = functools.partial(_sa_body, len(convs), K, r2, N, Mt, C1, C2)
    out = pl.pallas_call(
        body,
        grid=(B, M // Mt),
        in_specs=in_specs,
        out_specs=pl.BlockSpec((1, Mt, C2), lambda b, t: (b, t, 0)),
        out_shape=jax.ShapeDtypeStruct((B, M, C2), jnp.float32),
        scratch_shapes=[pltpu.VMEM((Mt, N), jnp.int32),
                        pltpu.VMEM((128, (N // 128) * C1), jnp.float32),
                        pltpu.VMEM((Mt, C2), jnp.float32)],
    )(*ins)
    return out


def _head_body(f_ref, mWT_ref, vWT_ref, bias_ref, out_ref):
    f = f_ref[...]
    C = f.shape[-1]
    m1 = jnp.sum(f, axis=-1) * (1.0 / C)
    diff = f - m1[:, :, None]
    v1 = jnp.sum(diff * diff, axis=-1) * (1.0 / (C - 1))
    out_ref[...] = (
        jnp.dot(m1, mWT_ref[...], preferred_element_type=jnp.float32)
        + jnp.dot(v1, vWT_ref[...], preferred_element_type=jnp.float32)
        + bias_ref[...])


def _head(feats, mWT, vWT, bias):
    B, M, C = feats.shape
    Z = mWT.shape[1]
    return pl.pallas_call(
        _head_body,
        out_shape=jax.ShapeDtypeStruct((B, Z), jnp.float32),
    )(feats, mWT, vWT, bias)


def kernel(x, params):
    xyz_nc = x
    xyz_cn = jnp.transpose(x, (0, 2, 1))
    feats = x
    for blk, ((cout, nlay), (m, r, k, mlp)) in zip(params["blocks"], _BLOCKS):
        N = xyz_nc.shape[1]
        stride = N // m
        ctr = xyz_nc[:, ::stride, :]
        convs = [(p["W"], p["b"]) for p in blk["conv"]]
        sa = blk["sa"]
        feats = _sa_block(xyz_nc, xyz_cn, feats, ctr, convs,
                          sa[0]["W"], sa[0]["b"], sa[1]["W"], sa[1]["b"],
                          m, k, r * r)
        xyz_nc = ctr
        xyz_cn = jnp.transpose(ctr, (0, 2, 1))
    bias = (params["mean_b"] + params["var_b"] + params["pe"]).reshape(1, -1)
    return _head(feats, params["mean_W"].T, params["var_W"].T, bias)


# speedup vs baseline: 1.2902x; 1.2902x over previous
import functools

import jax
import jax.numpy as jnp
from jax.experimental import pallas as pl
from jax.experimental.pallas import tpu as pltpu

_ZDIM = 128
_BLOCKS = [((32, 2), (1024, 0.1, 32, (32, 32))),
           ((32, 1), (256, 0.2, 32, (32, 64))),
           ((32, 1), (128, 0.4, 32, (64, _ZDIM)))]

_IMAX = 2147483647
_QBITS = 19


def _sa_body(nconv, K, r2, N, Mt, C1, C2, *refs):
    nw = 2 * nconv + 5
    (xyz_nc_ref, xyz_cn_ref, feats_ref, ctr_ref) = refs[:4]
    wrefs = refs[4:4 + nw]
    out_ref = refs[4 + nw]
    scores_ref = refs[5 + nw]
    a_ref = refs[6 + nw]
    acc_ref = refs[7 + nw]

    W1xT = wrefs[2 * nconv][...]
    b1 = wrefs[2 * nconv + 2][...]
    W2T = wrefs[2 * nconv + 3][...]
    b2 = wrefs[2 * nconv + 4][...]

    Nc = N // 128

    @pl.when(pl.program_id(1) == 0)
    def _compute_a():
        x_nc = xyz_nc_ref[0]
        f = feats_ref[0]
        for i in range(nconv):
            WcT = wrefs[2 * i][...]
            bc = wrefs[2 * i + 1][...]
            f = jnp.maximum(
                jnp.dot(f, WcT, preferred_element_type=jnp.float32) + bc, 0.0)
        W1fT = wrefs[2 * nconv + 1][...]
        a = (jnp.dot(x_nc, W1xT, preferred_element_type=jnp.float32)
             + jnp.dot(f, W1fT, preferred_element_type=jnp.float32)
             + b1)
        a_ref[...] = jnp.concatenate(
            [a[c * 128:(c + 1) * 128, :] for c in range(Nc)], axis=1)
    x_cn = xyz_cn_ref[0]
    ctr = ctr_ref[0]
    U = -jnp.dot(ctr, W1xT, preferred_element_type=jnp.float32)

    x2 = jnp.sum(x_cn * x_cn, axis=0, keepdims=True)
    c2 = jnp.sum(ctr * ctr, axis=1, keepdims=True)
    d2 = (c2 + x2
          - 2.0 * jnp.dot(ctr, x_cn, preferred_element_type=jnp.float32))

    iota = jax.lax.broadcasted_iota(jnp.int32, (Mt, N), 1)
    fbmin = jnp.min(d2, axis=1, keepdims=True)
    fb_amin = jnp.min(jnp.where(d2 == fbmin, iota, N), axis=1, keepdims=True)

    q = (d2 * (float(2 ** _QBITS) / r2)).astype(jnp.int32)
    q = jnp.minimum(jnp.maximum(q, 0), 2 ** _QBITS - 2)
    packed = q * 4096 + iota
    scores_ref[...] = jnp.where(d2 <= r2, packed, _IMAX)

    A2 = a_ref[...].astype(jnp.bfloat16)
    iota128 = jax.lax.broadcasted_iota(jnp.int32, (Mt, 128), 1)
    iotab = jax.lax.broadcasted_iota(jnp.int32, (Mt, Nc * C1), 1)
    l2c1 = C1.bit_length() - 1
    Rm = jnp.where(
        jnp.bitwise_and(
            jax.lax.broadcasted_iota(jnp.int32, (Nc * C1, C1), 0), C1 - 1)
        == jax.lax.broadcasted_iota(jnp.int32, (Nc * C1, C1), 1),
        1.0, 0.0).astype(jnp.bfloat16)
    acc_ref[...] = jnp.zeros((Mt, C2), jnp.float32)

    def cond(carry):
        j, _, alive = carry
        return jnp.logical_and(j < K, alive)

    def body(carry):
        j, prev, _ = carry
        s = scores_ref[...]
        vmin = jnp.min(jnp.where(s > prev, s, _IMAX), axis=1, keepdims=True)
        valid = vmin < _IMAX
        idx_sel = jnp.where(valid, jnp.bitwise_and(vmin, 4095), fb_amin)
        lane = jnp.bitwise_and(idx_sel, 127)
        chunk = jnp.right_shift(idx_sel, 7)
        onehotL = jnp.where(iota128 == lane, 1.0, 0.0).astype(jnp.bfloat16)
        B2 = jnp.dot(onehotL, A2, preferred_element_type=jnp.float32)
        mask2 = jnp.right_shift(iotab, l2c1) == chunk
        mB = jnp.where(mask2, B2, 0.0).astype(jnp.bfloat16)
        g = jnp.dot(mB, Rm, preferred_element_type=jnp.float32)
        h = jnp.maximum(g + U, 0.0)
        h2 = jnp.maximum(
            jnp.dot(h, W2T, preferred_element_type=jnp.float32) + b2, 0.0)
        acc_ref[...] = jnp.maximum(acc_ref[...], h2)
        return j + 1, vmin, jnp.min(vmin[:, 0]) < _IMAX

    jax.lax.while_loop(
        cond, body, (0, jnp.full((Mt, 1), -1, jnp.int32), True))
    out_ref[0] = acc_ref[...]


def _sa_block(xyz_nc, xyz_cn, feats, ctr, convs, W1, b1, W2, b2, M, K, r2):
    B, N, _ = xyz_nc.shape
    C = feats.shape[-1]
    C1 = W1.shape[0]
    C2 = W2.shape[0]
    Mt = min(M, 512)

    ins = [xyz_nc, xyz_cn, feats, ctr]
    in_specs = [
        pl.BlockSpec((1, N, 3), lambda b, t: (b, 0, 0)),
        pl.BlockSpec((1, 3, N), lambda b, t: (b, 0, 0)),
        pl.BlockSpec((1, N, C), lambda b, t: (b, 0, 0)),
        pl.BlockSpec((1, Mt, 3), lambda b, t: (b, t, 0)),
    ]
    weights = []
    for (Wc, bc) in convs:
        weights += [Wc.T, bc.reshape(1, -1)]
    weights += [W1[:, :3].T, W1[:, 3:].T, b1.reshape(1, -1),
                W2.T, b2.reshape(1, -1)]
    for w in weights:
        ins.append(w)
        in_specs.append(pl.BlockSpec(w.shape, lambda b, t: (0, 0)))

    body = functools.partial(_sa_body, len(convs), K, r2, N, Mt, C1, C2)
    out = pl.pallas_call(
        body,
        grid=(B, M // Mt),
        in_specs=in_specs,
        out_specs=pl.BlockSpec((1, Mt, C2), lambda b, t: (b, t, 0)),
        out_shape=jax.ShapeDtypeStruct((B, M, C2), jnp.float32),
        scratch_shapes=[pltpu.VMEM((Mt, N), jnp.int32),
                        pltpu.VMEM((128, (N // 128) * C1), jnp.float32),
                        pltpu.VMEM((Mt, C2), jnp.float32)],
    )(*ins)
    return out


def _head_body(f_ref, mWT_ref, vWT_ref, bias_ref, out_ref):
    f = f_ref[...]
    C = f.shape[-1]
    m1 = jnp.sum(f, axis=-1) * (1.0 / C)
    diff = f - m1[:, :, None]
    v1 = jnp.sum(diff * diff, axis=-1) * (1.0 / (C - 1))
    out_ref[...] = (
        jnp.dot(m1, mWT_ref[...], preferred_element_type=jnp.float32)
        + jnp.dot(v1, vWT_ref[...], preferred_element_type=jnp.float32)
        + bias_ref[...])


def _head(feats, mWT, vWT, bias):
    B, M, C = feats.shape
    Z = mWT.shape[1]
    return pl.pallas_call(
        _head_body,
        out_shape=jax.ShapeDtypeStruct((B, Z), jnp.float32),
    )(feats, mWT, vWT, bias)


def kernel(x, params):
    xyz_nc = x
    xyz_cn = jnp.transpose(x, (0, 2, 1))
    feats = x
    for blk, ((cout, nlay), (m, r, k, mlp)) in zip(params["blocks"], _BLOCKS):
        N = xyz_nc.shape[1]
        stride = N // m
        ctr = xyz_nc[:, ::stride, :]
        convs = [(p["W"], p["b"]) for p in blk["conv"]]
        sa = blk["sa"]
        feats = _sa_block(xyz_nc, xyz_cn, feats, ctr, convs,
                          sa[0]["W"], sa[0]["b"], sa[1]["W"], sa[1]["b"],
                          m, k, r * r)
        xyz_nc = ctr
        xyz_cn = jnp.transpose(ctr, (0, 2, 1))
    bias = (params["mean_b"] + params["var_b"] + params["pe"]).reshape(1, -1)
    return _head(feats, params["mean_W"].T, params["var_W"].T, bias)


# speedup vs baseline: 1.3632x; 1.0566x over previous
import functools

import jax
import jax.numpy as jnp
from jax.experimental import pallas as pl
from jax.experimental.pallas import tpu as pltpu

_ZDIM = 128
_BLOCKS = [((32, 2), (1024, 0.1, 32, (32, 32))),
           ((32, 1), (256, 0.2, 32, (32, 64))),
           ((32, 1), (128, 0.4, 32, (64, _ZDIM)))]

_IMAX = 2147483647
_QBITS = 19


def _sa_body(nconv, K, r2, N, Mt, C1, C2, *refs):
    nw = 2 * nconv + 5
    (xyz_nc_ref, xyz_cn_ref, feats_ref, ctr_ref) = refs[:4]
    wrefs = refs[4:4 + nw]
    out_ref = refs[4 + nw]
    scores_ref = refs[5 + nw]
    a_ref = refs[6 + nw]
    acc_ref = refs[7 + nw]

    W1xT = wrefs[2 * nconv][...]
    b1 = wrefs[2 * nconv + 2][...]
    W2T = wrefs[2 * nconv + 3][...]
    b2 = wrefs[2 * nconv + 4][...]

    Nc = N // 128

    @pl.when(pl.program_id(1) == 0)
    def _compute_a():
        x_nc = xyz_nc_ref[0]
        f = feats_ref[0]
        for i in range(nconv):
            WcT = wrefs[2 * i][...]
            bc = wrefs[2 * i + 1][...]
            f = jnp.maximum(
                jnp.dot(f, WcT, preferred_element_type=jnp.float32) + bc, 0.0)
        W1fT = wrefs[2 * nconv + 1][...]
        a = (jnp.dot(x_nc, W1xT, preferred_element_type=jnp.float32)
             + jnp.dot(f, W1fT, preferred_element_type=jnp.float32)
             + b1)
        a_ref[...] = jnp.concatenate(
            [a[c * 128:(c + 1) * 128, :] for c in range(Nc)], axis=1)
    x_cn = xyz_cn_ref[0]
    ctr = ctr_ref[0]
    U = -jnp.dot(ctr, W1xT, preferred_element_type=jnp.float32)

    x2 = jnp.sum(x_cn * x_cn, axis=0, keepdims=True)
    c2 = jnp.sum(ctr * ctr, axis=1, keepdims=True)
    d2 = (c2 + x2
          - 2.0 * jnp.dot(ctr, x_cn, preferred_element_type=jnp.float32))

    iota = jax.lax.broadcasted_iota(jnp.int32, (Mt, N), 1)
    fbmin = jnp.min(d2, axis=1, keepdims=True)
    fb_amin = jnp.min(jnp.where(d2 == fbmin, iota, N), axis=1, keepdims=True)

    q = (d2 * (float(2 ** _QBITS) / r2)).astype(jnp.int32)
    q = jnp.minimum(jnp.maximum(q, 0), 2 ** _QBITS - 2)
    packed = q * 4096 + iota
    scores_ref[...] = jnp.where(d2 <= r2, packed, _IMAX)

    A2 = a_ref[...].astype(jnp.bfloat16)
    iota128 = jax.lax.broadcasted_iota(jnp.int32, (Mt, 128), 1)
    iotab = jax.lax.broadcasted_iota(jnp.int32, (Mt, Nc * C1), 1)
    l2c1 = C1.bit_length() - 1
    Rm = jnp.where(
        jnp.bitwise_and(
            jax.lax.broadcasted_iota(jnp.int32, (Nc * C1, C1), 0), C1 - 1)
        == jax.lax.broadcasted_iota(jnp.int32, (Nc * C1, C1), 1),
        1.0, 0.0).astype(jnp.bfloat16)
    acc_ref[...] = jnp.zeros((Mt, C2), jnp.float32)

    def cond(carry):
        j, _, alive = carry
        return jnp.logical_and(j < K, alive)

    def body(carry):
        j, prev, _ = carry
        s = scores_ref[...]
        vmin = jnp.min(jnp.where(s > prev, s, _IMAX), axis=1, keepdims=True)
        valid = vmin < _IMAX
        idx_sel = jnp.where(valid, jnp.bitwise_and(vmin, 4095), fb_amin)
        lane = jnp.bitwise_and(idx_sel, 127)
        chunk = jnp.right_shift(idx_sel, 7)
        onehotL = jnp.where(iota128 == lane, 1.0, 0.0).astype(jnp.bfloat16)
        B2 = jnp.dot(onehotL, A2, preferred_element_type=jnp.float32)
        mask2 = jnp.right_shift(iotab, l2c1) == chunk
        mB = jnp.where(mask2, B2, 0.0).astype(jnp.bfloat16)
        g = jnp.dot(mB, Rm, preferred_element_type=jnp.float32)
        h = jnp.maximum(g + U, 0.0)
        h2 = jnp.maximum(
            jnp.dot(h, W2T, preferred_element_type=jnp.float32) + b2, 0.0)
        acc_ref[...] = jnp.maximum(acc_ref[...], h2)
        return j + 1, vmin, jnp.min(vmin[:, 0]) < _IMAX

    jax.lax.while_loop(
        cond, body, (0, jnp.full((Mt, 1), -1, jnp.int32), True))
    out_ref[0] = acc_ref[...]


def _sa_block(xyz_nc, xyz_cn, feats, ctr, convs, W1, b1, W2, b2, M, K, r2):
    B, N, _ = xyz_nc.shape
    C = feats.shape[-1]
    C1 = W1.shape[0]
    C2 = W2.shape[0]
    Mt = min(M, 1024)

    ins = [xyz_nc, xyz_cn, feats, ctr]
    in_specs = [
        pl.BlockSpec((1, N, 3), lambda b, t: (b, 0, 0)),
        pl.BlockSpec((1, 3, N), lambda b, t: (b, 0, 0)),
        pl.BlockSpec((1, N, C), lambda b, t: (b, 0, 0)),
        pl.BlockSpec((1, Mt, 3), lambda b, t: (b, t, 0)),
    ]
    weights = []
    for (Wc, bc) in convs:
        weights += [Wc.T, bc.reshape(1, -1)]
    weights += [W1[:, :3].T, W1[:, 3:].T, b1.reshape(1, -1),
                W2.T, b2.reshape(1, -1)]
    for w in weights:
        ins.append(w)
        in_specs.append(pl.BlockSpec(w.shape, lambda b, t: (0, 0)))

    body = functools.partial(_sa_body, len(convs), K, r2, N, Mt, C1, C2)
    out = pl.pallas_call(
        body,
        grid=(B, M // Mt),
        in_specs=in_specs,
        out_specs=pl.BlockSpec((1, Mt, C2), lambda b, t: (b, t, 0)),
        out_shape=jax.ShapeDtypeStruct((B, M, C2), jnp.float32),
        scratch_shapes=[pltpu.VMEM((Mt, N), jnp.int32),
                        pltpu.VMEM((128, (N // 128) * C1), jnp.float32),
                        pltpu.VMEM((Mt, C2), jnp.float32)],
    )(*ins)
    return out


def _head_body(f_ref, mWT_ref, vWT_ref, bias_ref, out_ref):
    f = f_ref[...]
    C = f.shape[-1]
    m1 = jnp.sum(f, axis=-1) * (1.0 / C)
    diff = f - m1[:, :, None]
    v1 = jnp.sum(diff * diff, axis=-1) * (1.0 / (C - 1))
    out_ref[...] = (
        jnp.dot(m1, mWT_ref[...], preferred_element_type=jnp.float32)
        + jnp.dot(v1, vWT_ref[...], preferred_element_type=jnp.float32)
        + bias_ref[...])


def _head(feats, mWT, vWT, bias):
    B, M, C = feats.shape
    Z = mWT.shape[1]
    return pl.pallas_call(
        _head_body,
        out_shape=jax.ShapeDtypeStruct((B, Z), jnp.float32),
    )(feats, mWT, vWT, bias)


def kernel(x, params):
    xyz_nc = x
    xyz_cn = jnp.transpose(x, (0, 2, 1))
    feats = x
    for blk, ((cout, nlay), (m, r, k, mlp)) in zip(params["blocks"], _BLOCKS):
        N = xyz_nc.shape[1]
        stride = N // m
        ctr = xyz_nc[:, ::stride, :]
        convs = [(p["W"], p["b"]) for p in blk["conv"]]
        sa = blk["sa"]
        feats = _sa_block(xyz_nc, xyz_cn, feats, ctr, convs,
                          sa[0]["W"], sa[0]["b"], sa[1]["W"], sa[1]["b"],
                          m, k, r * r)
        xyz_nc = ctr
        xyz_cn = jnp.transpose(ctr, (0, 2, 1))
    bias = (params["mean_b"] + params["var_b"] + params["pe"]).reshape(1, -1)
    return _head(feats, params["mean_W"].T, params["var_W"].T, bias)
